# baseline (speedup 1.0000x reference)
import functools

import jax
import jax.numpy as jnp
from jax import lax
from jax.experimental import pallas as pl
from jax.experimental.pallas import tpu as pltpu
from jax.experimental.pallas import tpu_sc as plsc

NC = 2
NS = 16
NW = NC * NS

N_P = 10240
ROWS_PER_TILE = N_P // NS
_NBUF = 10
_LEAD = 7
D1 = 64
D2 = 48


def _sc_mesh():
    return plsc.VectorSubcoreMesh(core_axis_name="c", subcore_axis_name="s")


_SC_PARAMS = pltpu.CompilerParams(use_tc_tiling_on_sc=False)


def _make_degree_kernel(m_deg, b_sz):
    nq = 8
    assert m_deg % nq == 0 and m_deg // nq >= 2

    @functools.partial(
        pl.kernel,
        out_type=jax.ShapeDtypeStruct((NC, N_P, 16), jnp.float32),
        mesh=_sc_mesh(),
        scratch_types=[
            pltpu.VMEM((m_deg, b_sz), jnp.int32),
            pltpu.VMEM((b_sz, 16), jnp.float32),
            pltpu.VMEM_SHARED((N_P, 16), jnp.float32),
            pltpu.SemaphoreType.DMA((nq,)),
        ],
        compiler_params=_SC_PARAMS,
    )
    def deg_kernel(e3_hbm, ones_hbm, zeros_hbm, out_hbm,
                   dst_v, ones_v, acc, ssem):
        c = lax.axis_index("c")
        s = lax.axis_index("s")
        w = c * NS + s
        row0 = s * ROWS_PER_TILE
        pltpu.sync_copy(zeros_hbm, acc.at[pl.ds(row0, ROWS_PER_TILE)])
        pltpu.sync_copy(ones_hbm, ones_v)
        pltpu.sync_copy(e3_hbm.at[1, pl.ds(w * m_deg, m_deg)], dst_v)
        plsc.subcore_barrier()

        def sstart(j, q):
            pltpu.async_copy(ones_v, acc.at[dst_v.at[j]], ssem.at[q],
                             add=True)

        def swait(j, q):
            pltpu.make_async_copy(
                ones_v, acc.at[dst_v.at[j]], ssem.at[q]).wait()

        for q in range(nq):
            sstart(q, q)

        def body(i, carry):
            for q in range(nq):
                j = i * nq + q
                swait(j - nq, q)
                sstart(j, q)
            return carry

        lax.fori_loop(1, m_deg // nq, body, 0)
        for q in range(nq):
            swait(m_deg - nq + q, q)
        plsc.subcore_barrier()
        pltpu.sync_copy(
            acc.at[pl.ds(row0, ROWS_PER_TILE)],
            out_hbm.at[c, pl.ds(row0, ROWS_PER_TILE)],
        )

    return deg_kernel


def _make_agg_kernel(m_chunks, b_sz, d_core):
    assert m_chunks % _NBUF == 0 and m_chunks // _NBUF >= 3

    @functools.partial(
        pl.kernel,
        out_type=jax.ShapeDtypeStruct((NC, N_P, d_core), jnp.float32),
        mesh=_sc_mesh(),
        scratch_types=[
            pltpu.VMEM((m_chunks, b_sz), jnp.int32),
            pltpu.VMEM((m_chunks, b_sz), jnp.int32),
            pltpu.VMEM((_NBUF, b_sz, d_core), jnp.float32),
            pltpu.VMEM_SHARED((N_P, d_core), jnp.float32),
            pltpu.SemaphoreType.DMA((_NBUF,)),
            pltpu.SemaphoreType.DMA((_NBUF,)),
        ],
        compiler_params=_SC_PARAMS,
    )
    def agg_kernel(g_hbm, e3_hbm, zeros_hbm, out_hbm,
                   src_v, dst_v, rows_v, acc, gsem, ssem):
        c = lax.axis_index("c")
        s = lax.axis_index("s")
        row0 = s * ROWS_PER_TILE
        tbl = g_hbm.at[c]
        pltpu.sync_copy(zeros_hbm, acc.at[pl.ds(row0, ROWS_PER_TILE)])
        pltpu.sync_copy(e3_hbm.at[0, pl.ds(s * m_chunks, m_chunks)], src_v)
        pltpu.sync_copy(e3_hbm.at[1, pl.ds(s * m_chunks, m_chunks)], dst_v)
        plsc.subcore_barrier()

        def gstart(j, b):
            pltpu.async_copy(tbl.at[src_v.at[j]], rows_v.at[b], gsem.at[b])

        def gwait(j, b):
            pltpu.make_async_copy(
                tbl.at[src_v.at[j]], rows_v.at[b], gsem.at[b]).wait()

        def sstart(j, b):
            pltpu.async_copy(rows_v.at[b], acc.at[dst_v.at[j]], ssem.at[b],
                             add=True)

        def swait(j, b):
            pltpu.make_async_copy(
                rows_v.at[b], acc.at[dst_v.at[j]], ssem.at[b]).wait()

        nb, ld = _NBUF, _LEAD
        lag = nb - ld
        nblk = m_chunks // nb

        def do_chunk(j, b, has_swait, has_gstart):
            fslot = (b + ld) % nb
            if has_swait:
                swait(j - lag, fslot)
            if has_gstart:
                gstart(j + ld, fslot)
            gwait(j, b)
            sstart(j, b)

        for j in range(ld):
            gstart(j, j)
        for b in range(nb):
            do_chunk(b, b, b >= lag, True)

        def body(i, carry):
            for b in range(nb):
                do_chunk(i * nb + b, b, True, True)
            return carry

        lax.fori_loop(1, nblk - 1, body, 0)
        for b in range(nb):
            j = (nblk - 1) * nb + b
            do_chunk(j, b, True, j + ld < m_chunks)
        for j in range(m_chunks - lag, m_chunks):
            swait(j, j % nb)
        plsc.subcore_barrier()
        pltpu.sync_copy(
            acc.at[pl.ds(row0, ROWS_PER_TILE)],
            out_hbm.at[c, pl.ds(row0, ROWS_PER_TILE)],
        )

    return agg_kernel


_GRID = 5
_BR = 2000


def _tc_mm_body(x_ref, w1_ref, h_ref):
    h_ref[...] = jnp.dot(x_ref[...], w1_ref[...],
                         preferred_element_type=jnp.float32)


def _tc1_body(h_ref, deg3_ref, g1_ref, s1_ref):
    deg = deg3_ref[0, :, 0:1] + deg3_ref[1, :, 0:1] + 1.0
    dis = lax.rsqrt(deg)
    h = h_ref[...]
    g1_ref[0] = h[:, 0:D1 // 2] * dis
    g1_ref[1] = h[:, D1 // 2:D1] * dis
    s1_ref[...] = h * (dis * dis)


def _tc2_body(a1_ref, deg3_ref, s1_ref, b1_ref, w2_ref, g2_ref, s2_ref):
    deg = deg3_ref[0, :, 0:1] + deg3_ref[1, :, 0:1] + 1.0
    dis = lax.rsqrt(deg)
    hd = D1 // 2
    s1 = s1_ref[...]
    b1r = b1_ref[0:1, :]
    z0 = jnp.maximum(dis * a1_ref[0] + s1[:, 0:hd] + b1r[:, 0:hd], 0.0)
    z1 = jnp.maximum(dis * a1_ref[1] + s1[:, hd:D1] + b1r[:, hd:D1], 0.0)
    h2 = (jnp.dot(z0, w2_ref[0:hd, :], preferred_element_type=jnp.float32)
          + jnp.dot(z1, w2_ref[hd:D1, :], preferred_element_type=jnp.float32))
    g2_ref[0] = h2[:, 0:D2 // 2] * dis
    g2_ref[1] = h2[:, D2 // 2:D2] * dis
    s2_ref[...] = h2[:, 0:40] * (dis * dis)


def _tc3_body(a2_ref, deg3_ref, s2_ref, b2_ref, out_ref):
    deg = deg3_ref[0, :, 0:1] + deg3_ref[1, :, 0:1] + 1.0
    dis = lax.rsqrt(deg)
    cd = D2 // 2
    s2 = s2_ref[...]
    b2r = b2_ref[0:1, :]
    out_ref[:, 0:cd] = dis * a2_ref[0] + s2[:, 0:cd] + b2r[:, 0:cd]
    out_ref[:, cd:40] = (dis * a2_ref[1][:, 0:40 - cd]
                         + s2[:, cd:40] + b2r[:, cd:40])


def _row_spec(d):
    return pl.BlockSpec((_BR, d), lambda i: (i, 0))


def _pair_spec(d):
    return pl.BlockSpec((NC, _BR, d), lambda i: (0, i, 0))


def _full_spec(shape):
    return pl.BlockSpec(shape, lambda i: tuple(0 for _ in shape))


def kernel(x, edge_index, W1, b1, W2, b2):
    n, f_in = x.shape
    hid = W1.shape[1]
    cls = W2.shape[1]
    e = edge_index.shape[1]

    b_sz = None
    for cand in range(128, 63, -1):
        m = e // (NS * cand)
        if (e == NS * cand * m and m % (2 * _NBUF) == 0
                and (m * cand) % 16 == 0 and (m // 2 * cand) % 16 == 0):
            b_sz = cand
            m_chunks = m
            break
    assert b_sz is not None, "no exact chunking for edge count"
    m_deg = m_chunks // 2

    e3 = edge_index.reshape(2, NS * m_chunks, b_sz)

    ones16 = jnp.ones((b_sz, 16), jnp.float32)
    zeros16 = jnp.zeros((ROWS_PER_TILE, 16), jnp.float32)
    zeros_d1 = jnp.zeros((ROWS_PER_TILE, D1 // 2), jnp.float32)
    zeros_d2 = jnp.zeros((ROWS_PER_TILE, D2 // 2), jnp.float32)

    deg_p = _make_degree_kernel(m_deg, b_sz)(e3, ones16, zeros16)

    h1 = pl.pallas_call(
        _tc_mm_body,
        grid=(_GRID,),
        in_specs=[_row_spec(f_in), _full_spec((f_in, hid))],
        out_specs=_row_spec(hid),
        out_shape=jax.ShapeDtypeStruct((n, hid), jnp.float32),
    )(x, W1)

    g1, s1 = pl.pallas_call(
        _tc1_body,
        grid=(_GRID,),
        in_specs=[
            _row_spec(hid),
            _pair_spec(16),
        ],
        out_specs=[_pair_spec(D1 // 2), _row_spec(hid)],
        out_shape=[
            jax.ShapeDtypeStruct((NC, N_P, D1 // 2), jnp.float32),
            jax.ShapeDtypeStruct((n, hid), jnp.float32),
        ],
    )(h1, deg_p)

    acc1 = _make_agg_kernel(m_chunks, b_sz, D1 // 2)(g1, e3, zeros_d1)

    b1f = jnp.broadcast_to(b1[None, :], (8, hid))
    w2p = jnp.zeros((hid, D2), jnp.float32).at[:, :cls].set(W2)
    g2, s2 = pl.pallas_call(
        _tc2_body,
        grid=(_GRID,),
        in_specs=[
            _pair_spec(D1 // 2),
            _pair_spec(16),
            _row_spec(hid),
            _full_spec((8, hid)),
            _full_spec((hid, D2)),
        ],
        out_specs=[_pair_spec(D2 // 2), _row_spec(cls)],
        out_shape=[
            jax.ShapeDtypeStruct((NC, N_P, D2 // 2), jnp.float32),
            jax.ShapeDtypeStruct((n, cls), jnp.float32),
        ],
    )(acc1, deg_p, s1, b1f, w2p)

    acc2 = _make_agg_kernel(m_chunks, b_sz, D2 // 2)(g2, e3, zeros_d2)

    b2f = jnp.broadcast_to(b2[None, :], (8, cls))
    out = pl.pallas_call(
        _tc3_body,
        grid=(_GRID,),
        in_specs=[
            _pair_spec(D2 // 2),
            _pair_spec(16),
            _row_spec(cls),
            _full_spec((8, cls)),
        ],
        out_specs=_row_spec(cls),
        out_shape=jax.ShapeDtypeStruct((n, cls), jnp.float32),
    )(acc2, deg_p, s2, b2f)

    return out


# speedup vs baseline: 1.0050x; 1.0050x over previous
import functools

import jax
import jax.numpy as jnp
from jax import lax
from jax.experimental import pallas as pl
from jax.experimental.pallas import tpu as pltpu
from jax.experimental.pallas import tpu_sc as plsc

NC = 2
NS = 16
NW = NC * NS

N_P = 10240
ROWS_PER_TILE = N_P // NS
_NBUF = 10
_LEAD = 7
D1 = 64
D2 = 48


def _sc_mesh():
    return plsc.VectorSubcoreMesh(core_axis_name="c", subcore_axis_name="s")


_SC_PARAMS = pltpu.CompilerParams(use_tc_tiling_on_sc=False)


def _make_degree_kernel(m_deg, b_sz):
    nq = 8
    assert m_deg % nq == 0 and m_deg // nq >= 2

    @functools.partial(
        pl.kernel,
        out_type=jax.ShapeDtypeStruct((NC, N_P, 16), jnp.float32),
        mesh=_sc_mesh(),
        scratch_types=[
            pltpu.VMEM((m_deg, b_sz), jnp.int32),
            pltpu.VMEM((b_sz, 16), jnp.float32),
            pltpu.VMEM_SHARED((N_P, 16), jnp.float32),
            pltpu.SemaphoreType.DMA((nq,)),
        ],
        compiler_params=_SC_PARAMS,
    )
    def deg_kernel(e3_hbm, ones_hbm, zeros_hbm, out_hbm,
                   dst_v, ones_v, acc, ssem):
        c = lax.axis_index("c")
        s = lax.axis_index("s")
        w = c * NS + s
        row0 = s * ROWS_PER_TILE
        pltpu.sync_copy(zeros_hbm, acc.at[pl.ds(row0, ROWS_PER_TILE)])
        pltpu.sync_copy(ones_hbm, ones_v)
        pltpu.sync_copy(e3_hbm.at[1, pl.ds(w * m_deg, m_deg)], dst_v)
        plsc.subcore_barrier()

        def sstart(j, q):
            pltpu.async_copy(ones_v, acc.at[dst_v.at[j]], ssem.at[q],
                             add=True)

        def swait(j, q):
            pltpu.make_async_copy(
                ones_v, acc.at[dst_v.at[j]], ssem.at[q]).wait()

        for q in range(nq):
            sstart(q, q)

        def body(i, carry):
            for q in range(nq):
                j = i * nq + q
                swait(j - nq, q)
                sstart(j, q)
            return carry

        lax.fori_loop(1, m_deg // nq, body, 0)
        for q in range(nq):
            swait(m_deg - nq + q, q)
        plsc.subcore_barrier()
        pltpu.sync_copy(
            acc.at[pl.ds(row0, ROWS_PER_TILE)],
            out_hbm.at[c, pl.ds(row0, ROWS_PER_TILE)],
        )

    return deg_kernel


def _make_agg_kernel(m_chunks, b_sz, d_core):
    assert m_chunks % _NBUF == 0 and m_chunks // _NBUF >= 3

    @functools.partial(
        pl.kernel,
        out_type=jax.ShapeDtypeStruct((NC, N_P, d_core), jnp.float32),
        mesh=_sc_mesh(),
        scratch_types=[
            pltpu.VMEM((m_chunks, b_sz), jnp.int32),
            pltpu.VMEM((m_chunks, b_sz), jnp.int32),
            pltpu.VMEM((_NBUF, b_sz, d_core), jnp.float32),
            pltpu.VMEM_SHARED((N_P, d_core), jnp.float32),
            pltpu.SemaphoreType.DMA((_NBUF,)),
            pltpu.SemaphoreType.DMA((_NBUF,)),
        ],
        compiler_params=_SC_PARAMS,
    )
    def agg_kernel(g_hbm, e3_hbm, zeros_hbm, out_hbm,
                   src_v, dst_v, rows_v, acc, gsem, ssem):
        c = lax.axis_index("c")
        s = lax.axis_index("s")
        row0 = s * ROWS_PER_TILE
        tbl = g_hbm.at[c]
        pltpu.sync_copy(zeros_hbm, acc.at[pl.ds(row0, ROWS_PER_TILE)])
        pltpu.sync_copy(e3_hbm.at[0, pl.ds(s * m_chunks, m_chunks)], src_v)
        pltpu.sync_copy(e3_hbm.at[1, pl.ds(s * m_chunks, m_chunks)], dst_v)
        plsc.subcore_barrier()

        def gstart(j, b):
            pltpu.async_copy(tbl.at[src_v.at[j]], rows_v.at[b], gsem.at[b])

        def gwait(j, b):
            pltpu.make_async_copy(
                tbl.at[src_v.at[j]], rows_v.at[b], gsem.at[b]).wait()

        def sstart(j, b):
            pltpu.async_copy(rows_v.at[b], acc.at[dst_v.at[j]], ssem.at[b],
                             add=True)

        def swait(j, b):
            pltpu.make_async_copy(
                rows_v.at[b], acc.at[dst_v.at[j]], ssem.at[b]).wait()

        nb, ld = _NBUF, _LEAD
        lag = nb - ld
        nblk = m_chunks // nb

        def do_chunk(j, b, has_swait, has_gstart):
            fslot = (b + ld) % nb
            if has_swait:
                swait(j - lag, fslot)
            if has_gstart:
                gstart(j + ld, fslot)
            gwait(j, b)
            sstart(j, b)

        for j in range(ld):
            gstart(j, j)
        for b in range(nb):
            do_chunk(b, b, b >= lag, True)

        def body(i, carry):
            for b in range(nb):
                do_chunk(i * nb + b, b, True, True)
            return carry

        lax.fori_loop(1, nblk - 1, body, 0)
        for b in range(nb):
            j = (nblk - 1) * nb + b
            do_chunk(j, b, True, j + ld < m_chunks)
        for j in range(m_chunks - lag, m_chunks):
            swait(j, j % nb)
        plsc.subcore_barrier()
        pltpu.sync_copy(
            acc.at[pl.ds(row0, ROWS_PER_TILE)],
            out_hbm.at[c, pl.ds(row0, ROWS_PER_TILE)],
        )

    return agg_kernel


_GRID = 5
_BR = 2000


def _tc_mm_body(x_ref, w1_ref, h_ref):
    h_ref[...] = jnp.dot(x_ref[...], w1_ref[...],
                         preferred_element_type=jnp.float32)


def _tc1_body(h_ref, deg3_ref, g1_ref, s1_ref):
    deg = deg3_ref[0, :, 0:1] + deg3_ref[1, :, 0:1] + 1.0
    dis = lax.rsqrt(deg)
    h = h_ref[...]
    g1_ref[0] = h[:, 0:D1 // 2] * dis
    g1_ref[1] = h[:, D1 // 2:D1] * dis
    s1_ref[...] = h * (dis * dis)


def _tc2_body(a1_ref, deg3_ref, s1_ref, b1_ref, w2_ref, g2_ref, s2_ref):
    deg = deg3_ref[0, :, 0:1] + deg3_ref[1, :, 0:1] + 1.0
    dis = lax.rsqrt(deg)
    agg = jnp.concatenate([a1_ref[0], a1_ref[1]], axis=1)
    z = dis * agg + s1_ref[...] + b1_ref[0:1, :]
    z = jnp.maximum(z, 0.0)
    h2 = jnp.dot(z, w2_ref[...], preferred_element_type=jnp.float32)
    g2_ref[0] = h2[:, 0:D2 // 2] * dis
    g2_ref[1] = h2[:, D2 // 2:D2] * dis
    s2_ref[...] = h2[:, 0:40] * (dis * dis)


def _tc3_body(a2_ref, deg3_ref, s2_ref, b2_ref, out_ref):
    deg = deg3_ref[0, :, 0:1] + deg3_ref[1, :, 0:1] + 1.0
    dis = lax.rsqrt(deg)
    agg = jnp.concatenate([a2_ref[0], a2_ref[1][:, 0:40 - D2 // 2]], axis=1)
    out_ref[...] = dis * agg + s2_ref[...] + b2_ref[0:1, :]


def _row_spec(d):
    return pl.BlockSpec((_BR, d), lambda i: (i, 0))


def _pair_spec(d):
    return pl.BlockSpec((NC, _BR, d), lambda i: (0, i, 0))


def _full_spec(shape):
    return pl.BlockSpec(shape, lambda i: tuple(0 for _ in shape))


def kernel(x, edge_index, W1, b1, W2, b2):
    n, f_in = x.shape
    hid = W1.shape[1]
    cls = W2.shape[1]
    e = edge_index.shape[1]

    b_sz = None
    for cand in range(128, 63, -1):
        m = e // (NS * cand)
        if (e == NS * cand * m and m % (2 * _NBUF) == 0
                and (m * cand) % 16 == 0 and (m // 2 * cand) % 16 == 0):
            b_sz = cand
            m_chunks = m
            break
    assert b_sz is not None, "no exact chunking for edge count"
    m_deg = m_chunks // 2

    e3 = edge_index.reshape(2, NS * m_chunks, b_sz)

    ones16 = jnp.ones((b_sz, 16), jnp.float32)
    zeros16 = jnp.zeros((ROWS_PER_TILE, 16), jnp.float32)
    zeros_d1 = jnp.zeros((ROWS_PER_TILE, D1 // 2), jnp.float32)
    zeros_d2 = jnp.zeros((ROWS_PER_TILE, D2 // 2), jnp.float32)

    deg_p = _make_degree_kernel(m_deg, b_sz)(e3, ones16, zeros16)

    h1 = pl.pallas_call(
        _tc_mm_body,
        grid=(_GRID,),
        in_specs=[_row_spec(f_in), _full_spec((f_in, hid))],
        out_specs=_row_spec(hid),
        out_shape=jax.ShapeDtypeStruct((n, hid), jnp.float32),
    )(x, W1)

    g1, s1 = pl.pallas_call(
        _tc1_body,
        grid=(_GRID,),
        in_specs=[
            _row_spec(hid),
            _pair_spec(16),
        ],
        out_specs=[_pair_spec(D1 // 2), _row_spec(hid)],
        out_shape=[
            jax.ShapeDtypeStruct((NC, N_P, D1 // 2), jnp.float32),
            jax.ShapeDtypeStruct((n, hid), jnp.float32),
        ],
    )(h1, deg_p)

    acc1 = _make_agg_kernel(m_chunks, b_sz, D1 // 2)(g1, e3, zeros_d1)

    b1f = jnp.broadcast_to(b1[None, :], (8, hid))
    w2p = jnp.zeros((hid, D2), jnp.float32).at[:, :cls].set(W2)
    g2, s2 = pl.pallas_call(
        _tc2_body,
        grid=(_GRID,),
        in_specs=[
            _pair_spec(D1 // 2),
            _pair_spec(16),
            _row_spec(hid),
            _full_spec((8, hid)),
            _full_spec((hid, D2)),
        ],
        out_specs=[_pair_spec(D2 // 2), _row_spec(cls)],
        out_shape=[
            jax.ShapeDtypeStruct((NC, N_P, D2 // 2), jnp.float32),
            jax.ShapeDtypeStruct((n, cls), jnp.float32),
        ],
    )(acc1, deg_p, s1, b1f, w2p)

    acc2 = _make_agg_kernel(m_chunks, b_sz, D2 // 2)(g2, e3, zeros_d2)

    b2f = jnp.broadcast_to(b2[None, :], (8, cls))
    out = pl.pallas_call(
        _tc3_body,
        grid=(_GRID,),
        in_specs=[
            _pair_spec(D2 // 2),
            _pair_spec(16),
            _row_spec(cls),
            _full_spec((8, cls)),
        ],
        out_specs=_row_spec(cls),
        out_shape=jax.ShapeDtypeStruct((n, cls), jnp.float32),
    )(acc2, deg_p, s2, b2f)

    return out


# speedup vs baseline: 1.0280x; 1.0229x over previous
import functools

import jax
import jax.numpy as jnp
from jax import lax
from jax.experimental import pallas as pl
from jax.experimental.pallas import tpu as pltpu
from jax.experimental.pallas import tpu_sc as plsc

NC = 2
NS = 16
NW = NC * NS

N_P = 10240
ROWS_PER_TILE = N_P // NS
_NBUF = 10
_LEAD = 7
D1 = 64
D2 = 48


def _sc_mesh():
    return plsc.VectorSubcoreMesh(core_axis_name="c", subcore_axis_name="s")


_SC_PARAMS = pltpu.CompilerParams(use_tc_tiling_on_sc=False)


def _make_degree_kernel(m_deg, b_sz):
    nq = 8
    assert m_deg % nq == 0 and m_deg // nq >= 2

    @functools.partial(
        pl.kernel,
        out_type=jax.ShapeDtypeStruct((NC, N_P, 8), jnp.float32),
        mesh=_sc_mesh(),
        scratch_types=[
            pltpu.VMEM((m_deg, b_sz), jnp.int32),
            pltpu.VMEM((b_sz, 8), jnp.float32),
            pltpu.VMEM_SHARED((N_P, 8), jnp.float32),
            pltpu.SemaphoreType.DMA((nq,)),
        ],
        compiler_params=_SC_PARAMS,
    )
    def deg_kernel(e3_hbm, ones_hbm, zeros_hbm, out_hbm,
                   dst_v, ones_v, acc, ssem):
        c = lax.axis_index("c")
        s = lax.axis_index("s")
        w = c * NS + s
        row0 = s * ROWS_PER_TILE
        pltpu.sync_copy(zeros_hbm, acc.at[pl.ds(row0, ROWS_PER_TILE)])
        pltpu.sync_copy(ones_hbm, ones_v)
        pltpu.sync_copy(e3_hbm.at[1, pl.ds(w * m_deg, m_deg)], dst_v)
        plsc.subcore_barrier()

        def sstart(j, q):
            pltpu.async_copy(ones_v, acc.at[dst_v.at[j]], ssem.at[q],
                             add=True)

        def swait(j, q):
            pltpu.make_async_copy(
                ones_v, acc.at[dst_v.at[j]], ssem.at[q]).wait()

        for q in range(nq):
            sstart(q, q)

        def body(i, carry):
            for q in range(nq):
                j = i * nq + q
                swait(j - nq, q)
                sstart(j, q)
            return carry

        lax.fori_loop(1, m_deg // nq, body, 0)
        for q in range(nq):
            swait(m_deg - nq + q, q)
        plsc.subcore_barrier()
        pltpu.sync_copy(
            acc.at[pl.ds(row0, ROWS_PER_TILE)],
            out_hbm.at[c, pl.ds(row0, ROWS_PER_TILE)],
        )

    return deg_kernel


def _make_agg_kernel(m_chunks, b_sz, d_core):
    assert m_chunks % _NBUF == 0 and m_chunks // _NBUF >= 3

    @functools.partial(
        pl.kernel,
        out_type=jax.ShapeDtypeStruct((NC, N_P, d_core), jnp.float32),
        mesh=_sc_mesh(),
        scratch_types=[
            pltpu.VMEM((m_chunks, b_sz), jnp.int32),
            pltpu.VMEM((m_chunks, b_sz), jnp.int32),
            pltpu.VMEM((_NBUF, b_sz, d_core), jnp.float32),
            pltpu.VMEM_SHARED((N_P, d_core), jnp.float32),
            pltpu.SemaphoreType.DMA((_NBUF,)),
            pltpu.SemaphoreType.DMA((_NBUF,)),
        ],
        compiler_params=_SC_PARAMS,
    )
    def agg_kernel(g_hbm, e3_hbm, zeros_hbm, out_hbm,
                   src_v, dst_v, rows_v, acc, gsem, ssem):
        c = lax.axis_index("c")
        s = lax.axis_index("s")
        row0 = s * ROWS_PER_TILE
        tbl = g_hbm.at[c]
        pltpu.sync_copy(zeros_hbm, acc.at[pl.ds(row0, ROWS_PER_TILE)])
        pltpu.sync_copy(e3_hbm.at[0, pl.ds(s * m_chunks, m_chunks)], src_v)
        pltpu.sync_copy(e3_hbm.at[1, pl.ds(s * m_chunks, m_chunks)], dst_v)
        plsc.subcore_barrier()

        def gstart(j, b):
            pltpu.async_copy(tbl.at[src_v.at[j]], rows_v.at[b], gsem.at[b])

        def gwait(j, b):
            pltpu.make_async_copy(
                tbl.at[src_v.at[j]], rows_v.at[b], gsem.at[b]).wait()

        def sstart(j, b):
            pltpu.async_copy(rows_v.at[b], acc.at[dst_v.at[j]], ssem.at[b],
                             add=True)

        def swait(j, b):
            pltpu.make_async_copy(
                rows_v.at[b], acc.at[dst_v.at[j]], ssem.at[b]).wait()

        nb, ld = _NBUF, _LEAD
        lag = nb - ld
        nblk = m_chunks // nb

        def do_chunk(j, b, has_swait, has_gstart):
            fslot = (b + ld) % nb
            if has_swait:
                swait(j - lag, fslot)
            if has_gstart:
                gstart(j + ld, fslot)
            gwait(j, b)
            sstart(j, b)

        for j in range(ld):
            gstart(j, j)
        for b in range(nb):
            do_chunk(b, b, b >= lag, True)

        def body(i, carry):
            for b in range(nb):
                do_chunk(i * nb + b, b, True, True)
            return carry

        lax.fori_loop(1, nblk - 1, body, 0)
        for b in range(nb):
            j = (nblk - 1) * nb + b
            do_chunk(j, b, True, j + ld < m_chunks)
        for j in range(m_chunks - lag, m_chunks):
            swait(j, j % nb)
        plsc.subcore_barrier()
        pltpu.sync_copy(
            acc.at[pl.ds(row0, ROWS_PER_TILE)],
            out_hbm.at[c, pl.ds(row0, ROWS_PER_TILE)],
        )

    return agg_kernel


_GRID = 5
_BR = 2000


def _tc_mm_body(x_ref, w1_ref, h_ref):
    h_ref[...] = jnp.dot(x_ref[...], w1_ref[...],
                         preferred_element_type=jnp.float32)


def _tc1_body(h_ref, deg3_ref, g1_ref, s1_ref):
    deg = deg3_ref[0, :, 0:1] + deg3_ref[1, :, 0:1] + 1.0
    dis = lax.rsqrt(deg)
    h = h_ref[...]
    g1_ref[0] = h[:, 0:D1 // 2] * dis
    g1_ref[1] = h[:, D1 // 2:D1] * dis
    s1_ref[...] = h * (dis * dis)


def _tc2_body(a1_ref, deg3_ref, s1_ref, b1_ref, w2_ref, g2_ref, s2_ref):
    deg = deg3_ref[0, :, 0:1] + deg3_ref[1, :, 0:1] + 1.0
    dis = lax.rsqrt(deg)
    agg = jnp.concatenate([a1_ref[0], a1_ref[1]], axis=1)
    z = dis * agg + s1_ref[...] + b1_ref[0:1, :]
    z = jnp.maximum(z, 0.0)
    h2 = jnp.dot(z, w2_ref[...], preferred_element_type=jnp.float32)
    g2_ref[0] = h2[:, 0:D2 // 2] * dis
    g2_ref[1] = h2[:, D2 // 2:D2] * dis
    s2_ref[...] = h2[:, 0:40] * (dis * dis)


def _tc3_body(a2_ref, deg3_ref, s2_ref, b2_ref, out_ref):
    deg = deg3_ref[0, :, 0:1] + deg3_ref[1, :, 0:1] + 1.0
    dis = lax.rsqrt(deg)
    agg = jnp.concatenate([a2_ref[0], a2_ref[1][:, 0:40 - D2 // 2]], axis=1)
    out_ref[...] = dis * agg + s2_ref[...] + b2_ref[0:1, :]


def _row_spec(d):
    return pl.BlockSpec((_BR, d), lambda i: (i, 0))


def _pair_spec(d):
    return pl.BlockSpec((NC, _BR, d), lambda i: (0, i, 0))


def _full_spec(shape):
    return pl.BlockSpec(shape, lambda i: tuple(0 for _ in shape))


def kernel(x, edge_index, W1, b1, W2, b2):
    n, f_in = x.shape
    hid = W1.shape[1]
    cls = W2.shape[1]
    e = edge_index.shape[1]

    b_sz = None
    for cand in range(128, 63, -1):
        m = e // (NS * cand)
        if (e == NS * cand * m and m % (2 * _NBUF) == 0
                and (m * cand) % 16 == 0 and (m // 2 * cand) % 16 == 0):
            b_sz = cand
            m_chunks = m
            break
    assert b_sz is not None, "no exact chunking for edge count"
    m_deg = m_chunks // 2

    e3 = edge_index.reshape(2, NS * m_chunks, b_sz)

    ones16 = jnp.ones((b_sz, 8), jnp.float32)
    zeros16 = jnp.zeros((ROWS_PER_TILE, 8), jnp.float32)
    zeros_d1 = jnp.zeros((ROWS_PER_TILE, D1 // 2), jnp.float32)
    zeros_d2 = jnp.zeros((ROWS_PER_TILE, D2 // 2), jnp.float32)

    deg_p = _make_degree_kernel(m_deg, b_sz)(e3, ones16, zeros16)

    h1 = pl.pallas_call(
        _tc_mm_body,
        grid=(_GRID,),
        in_specs=[_row_spec(f_in), _full_spec((f_in, hid))],
        out_specs=_row_spec(hid),
        out_shape=jax.ShapeDtypeStruct((n, hid), jnp.float32),
    )(x, W1)

    g1, s1 = pl.pallas_call(
        _tc1_body,
        grid=(_GRID,),
        in_specs=[
            _row_spec(hid),
            _pair_spec(8),
        ],
        out_specs=[_pair_spec(D1 // 2), _row_spec(hid)],
        out_shape=[
            jax.ShapeDtypeStruct((NC, N_P, D1 // 2), jnp.float32),
            jax.ShapeDtypeStruct((n, hid), jnp.float32),
        ],
    )(h1, deg_p)

    acc1 = _make_agg_kernel(m_chunks, b_sz, D1 // 2)(g1, e3, zeros_d1)

    b1f = jnp.broadcast_to(b1[None, :], (8, hid))
    w2p = jnp.zeros((hid, D2), jnp.float32).at[:, :cls].set(W2)
    g2, s2 = pl.pallas_call(
        _tc2_body,
        grid=(_GRID,),
        in_specs=[
            _pair_spec(D1 // 2),
            _pair_spec(8),
            _row_spec(hid),
            _full_spec((8, hid)),
            _full_spec((hid, D2)),
        ],
        out_specs=[_pair_spec(D2 // 2), _row_spec(cls)],
        out_shape=[
            jax.ShapeDtypeStruct((NC, N_P, D2 // 2), jnp.float32),
            jax.ShapeDtypeStruct((n, cls), jnp.float32),
        ],
    )(acc1, deg_p, s1, b1f, w2p)

    acc2 = _make_agg_kernel(m_chunks, b_sz, D2 // 2)(g2, e3, zeros_d2)

    b2f = jnp.broadcast_to(b2[None, :], (8, cls))
    out = pl.pallas_call(
        _tc3_body,
        grid=(_GRID,),
        in_specs=[
            _pair_spec(D2 // 2),
            _pair_spec(8),
            _row_spec(cls),
            _full_spec((8, cls)),
        ],
        out_specs=_row_spec(cls),
        out_shape=jax.ShapeDtypeStruct((n, cls), jnp.float32),
    )(acc2, deg_p, s2, b2f)

    return out
